# baseline (speedup 1.0000x reference)
import functools

import jax
import jax.numpy as jnp
import numpy as np
from jax import lax
from jax.experimental import pallas as pl
from jax.experimental.pallas import tpu as pltpu
from jax.experimental.pallas import tpu_sc as plsc

N_NODES = 10000
N_EDGES = 320000
D_FEAT = 128
D_EDGE = 16
D_OUT = 128
D_HALF = D_OUT // 2

NC = 2
NS = 16
NW = NC * NS
E_PER_W = N_EDGES // NW
CHUNK = 40
N_CHUNKS = E_PER_W // CHUNK
FILL = 2000

_HI_MASK = np.int32(-65536)


def _round_bits(v):
    return lax.bitcast_convert_type(
        v.astype(jnp.bfloat16).astype(jnp.float32), jnp.int32)


def _pack(ve, vo):
    return lax.shift_right_logical(_round_bits(ve), 16) | (
        _round_bits(vo) & _HI_MASK)


def _tc_body(x_ref, w1e_ref, w1o_ref, w3e_ref, w3o_ref, be_ref, bo_ref,
             eattr_ref, w2_ref, p1_ref, p3_ref, ea_ref):
    @pl.when(pl.program_id(0) == 0)
    def _():
        xb = x_ref[...]
        p1_ref[...] = _pack(
            jnp.dot(xb, w1e_ref[...], preferred_element_type=jnp.float32)
            + be_ref[...],
            jnp.dot(xb, w1o_ref[...], preferred_element_type=jnp.float32)
            + bo_ref[...],
        )
        p3_ref[...] = _pack(
            jnp.dot(xb, w3e_ref[...], preferred_element_type=jnp.float32),
            jnp.dot(xb, w3o_ref[...], preferred_element_type=jnp.float32),
        )

    ea_ref[...] = jnp.dot(
        eattr_ref[...], w2_ref[...], preferred_element_type=jnp.float32)


def _tc_stage(x, w1e, w1o, w3e, w3o, b_e, b_o, edge_attr, w2):
    grid = 40
    blk = N_EDGES // grid
    whole = lambda: pl.BlockSpec((D_FEAT, D_HALF), lambda i: (0, 0))
    bias = lambda: pl.BlockSpec((1, D_HALF), lambda i: (0, 0))
    return pl.pallas_call(
        _tc_body,
        grid=(grid,),
        in_specs=[
            pl.BlockSpec((N_NODES, D_FEAT), lambda i: (0, 0)),
            whole(), whole(), whole(), whole(), bias(), bias(),
            pl.BlockSpec((blk, D_EDGE), lambda i: (i, 0)),
            pl.BlockSpec((D_EDGE, D_OUT), lambda i: (0, 0)),
        ],
        out_specs=[
            pl.BlockSpec((N_NODES, D_HALF), lambda i: (0, 0)),
            pl.BlockSpec((N_NODES, D_HALF), lambda i: (0, 0)),
            pl.BlockSpec((blk, D_OUT), lambda i: (i, 0)),
        ],
        out_shape=[
            jax.ShapeDtypeStruct((N_NODES, D_HALF), jnp.int32),
            jax.ShapeDtypeStruct((N_NODES, D_HALF), jnp.int32),
            jax.ShapeDtypeStruct((N_EDGES, D_OUT), jnp.float32),
        ],
    )(x, w1e, w1o, w3e, w3o, b_e, b_o, edge_attr, w2)


def _sc_body(p1_hbm, p3_hbm, src_hbm, dst_hbm, ea_hbm, out_hbm,
             p1_sh, p3_sh,
             i1_0, i1_1, i1_2, i1_3, i3_0, i3_1, i3_2, i3_3,
             g1_0, g1_1, g3_0, g3_1, acc_0, acc_1, ob_0, ob_1,
             gsem0, gsem1, easem0, easem1, osem0, osem1,
             isem0, isem1, isem2, isem3):
    sid = lax.axis_index("s")
    wid = sid * NC + lax.axis_index("c")
    w_base = wid * E_PER_W

    i1 = (i1_0, i1_1, i1_2, i1_3)
    i3 = (i3_0, i3_1, i3_2, i3_3)
    isem = (isem0, isem1, isem2, isem3)
    g1 = (g1_0, g1_1)
    g3 = (g3_0, g3_1)
    acc = (acc_0, acc_1)
    ob = (ob_0, ob_1)
    gsem = (gsem0, gsem1)
    easem = (easem0, easem1)
    osem = (osem0, osem1)

    @pl.when(sid < 5)
    def _():
        fsl = pl.ds(sid * FILL, FILL)
        pltpu.sync_copy(p1_hbm.at[fsl], p1_sh.at[fsl])

    @pl.when((sid >= 5) & (sid < 10))
    def _():
        fsl = pl.ds((sid - 5) * FILL, FILL)
        pltpu.sync_copy(p3_hbm.at[fsl], p3_sh.at[fsl])

    plsc.subcore_barrier()

    def idx_descs(c, k):
        base = w_base + c * CHUNK
        return (
            pltpu.make_async_copy(
                src_hbm.at[pl.ds(base, CHUNK)], i1[k], isem[k]),
            pltpu.make_async_copy(
                dst_hbm.at[pl.ds(base, CHUNK)], i3[k], isem[k]),
        )

    def idx_start(c, k):
        for d in idx_descs(c, k):
            d.start()

    def idx_wait(c, k):
        for d in idx_descs(c, k):
            d.wait()

    def in_descs(c, b, k):
        base = w_base + c * CHUNK
        return (
            pltpu.make_async_copy(p1_sh.at[i1[k]], g1[b], gsem[b]),
            pltpu.make_async_copy(p3_sh.at[i3[k]], g3[b], gsem[b]),
            pltpu.make_async_copy(
                ea_hbm.at[pl.ds(base * D_OUT, CHUNK * D_OUT)], acc[b],
                easem[b]),
        )

    def out_desc(c, b):
        base = (w_base + c * CHUNK) * D_OUT
        return pltpu.make_async_copy(
            ob[b], out_hbm.at[pl.ds(base, CHUNK * D_OUT)], osem[b])

    def issue(c, b, k):
        for d in in_descs(c, b, k):
            d.start()

    def wait_in(c, b, k):
        for d in in_descs(c, b, k):
            d.wait()

    def _lo(w):
        return lax.bitcast_convert_type(w << 16, jnp.float32)

    def _hi(w):
        return lax.bitcast_convert_type(w & _HI_MASK, jnp.float32)

    def compute(b):
        def row_body(r, _):
            rbase = r * D_OUT
            for j in range(4):
                sl = pl.ds(j * 16, 16)
                w1v = g1[b][r, sl]
                w3v = g3[b][r, sl]
                ob[b][pl.ds(rbase + j * 16, 16)] = (
                    _lo(w1v) + _lo(w3v) + acc[b][pl.ds(rbase + j * 16, 16)])
                ob[b][pl.ds(rbase + D_HALF + j * 16, 16)] = (
                    _hi(w1v) + _hi(w3v)
                    + acc[b][pl.ds(rbase + D_HALF + j * 16, 16)])
            return 0

        lax.fori_loop(0, CHUNK, row_body, 0)

    for m in range(4):
        idx_start(m, m)
    idx_wait(0, 0)
    issue(0, 0, 0)
    idx_wait(1, 1)
    issue(1, 1, 1)
    for c0 in (0, 1):
        wait_in(c0, c0, c0)
        compute(c0)
        idx_start(c0 + 4, c0)
        idx_wait(c0 + 2, c0 + 2)
        issue(c0 + 2, c0, c0 + 2)
        out_desc(c0, c0).start()

    def quad_body(jq, _):
        cb = 2 + 4 * jq
        for off in range(4):
            c = cb + off
            k = (2 + off) % 4
            b = off % 2
            wait_in(c, b, k)
            out_desc(c - 2, b).wait()
            compute(b)

            @pl.when(c + 2 < N_CHUNKS)
            def _():
                @pl.when(c + 4 < N_CHUNKS)
                def _():
                    idx_start(c + 4, k)

                idx_wait(c + 2, (k + 2) % 4)
                issue(c + 2, b, (k + 2) % 4)

            out_desc(c, b).start()
        return 0

    lax.fori_loop(0, (N_CHUNKS - 2) // 4, quad_body, 0)

    out_desc(N_CHUNKS - 2, 0).wait()
    out_desc(N_CHUNKS - 1, 1).wait()


def _sc_combine(p1, p3, src, dst, ea):
    mesh = plsc.VectorSubcoreMesh(core_axis_name="c", subcore_axis_name="s")
    iblk = lambda: pltpu.VMEM((CHUNK,), jnp.int32)
    gblk = lambda: pltpu.VMEM((CHUNK, D_HALF), jnp.int32)
    eblk = lambda: pltpu.VMEM((CHUNK * D_OUT,), jnp.float32)
    outblk = lambda: pltpu.VMEM((CHUNK * D_OUT,), jnp.float32)
    f = functools.partial(
        pl.kernel,
        mesh=mesh,
        compiler_params=pltpu.CompilerParams(use_tc_tiling_on_sc=False),
        out_type=jax.ShapeDtypeStruct((N_EDGES * D_OUT,), jnp.float32),
        scratch_types=[
            pltpu.VMEM_SHARED((N_NODES, D_HALF), jnp.int32),
            pltpu.VMEM_SHARED((N_NODES, D_HALF), jnp.int32),
            iblk(), iblk(), iblk(), iblk(),
            iblk(), iblk(), iblk(), iblk(),
            gblk(), gblk(), gblk(), gblk(),
            eblk(), eblk(),
            outblk(), outblk(),
            pltpu.SemaphoreType.DMA,
            pltpu.SemaphoreType.DMA,
            pltpu.SemaphoreType.DMA,
            pltpu.SemaphoreType.DMA,
            pltpu.SemaphoreType.DMA,
            pltpu.SemaphoreType.DMA,
            pltpu.SemaphoreType.DMA,
            pltpu.SemaphoreType.DMA,
            pltpu.SemaphoreType.DMA,
            pltpu.SemaphoreType.DMA,
        ],
    )(_sc_body)
    return f(p1, p3, src, dst, ea.reshape(N_EDGES * D_OUT))


@jax.jit
def kernel(x, edge_attr, edge_index, W, b):
    w1 = W[:D_FEAT]
    w2 = W[D_FEAT:D_FEAT + D_EDGE]
    w3 = W[D_FEAT + D_EDGE:]
    p1, p3, ea = _tc_stage(
        x, w1[:, :D_HALF], w1[:, D_HALF:], w3[:, :D_HALF], w3[:, D_HALF:],
        b[:D_HALF].reshape(1, D_HALF), b[D_HALF:].reshape(1, D_HALF),
        edge_attr, w2)
    out = _sc_combine(p1, p3, edge_index[0], edge_index[1], ea)
    return out.reshape(N_EDGES, D_OUT)


# speedup vs baseline: 1.5540x; 1.5540x over previous
import functools

import jax
import jax.numpy as jnp
from jax import lax
from jax.experimental import pallas as pl
from jax.experimental.pallas import tpu as pltpu
from jax.experimental.pallas import tpu_sc as plsc

N_NODES = 10000
N_EDGES = 320000
D_FEAT = 128
D_EDGE = 16
D_OUT = 128

NC = 2
NS = 16
NW = NC * NS
E_PER_W = N_EDGES // NW
CHUNK = 80
N_CHUNKS = E_PER_W // CHUNK


def _tc_body(x_ref, w1_ref, w3_ref, b_ref, eattr_ref, w2_ref,
             p1_ref, p3_ref, ea_ref):
    @pl.when(pl.program_id(0) == 0)
    def _():
        xb = x_ref[...]
        p1_ref[...] = (
            jnp.dot(xb, w1_ref[...], preferred_element_type=jnp.float32)
            + b_ref[...]
        )
        p3_ref[...] = jnp.dot(
            xb, w3_ref[...], preferred_element_type=jnp.float32)

    ea_ref[...] = jnp.dot(
        eattr_ref[...], w2_ref[...], preferred_element_type=jnp.float32)


def _tc_stage(x, w1, w3, b2d, edge_attr, w2):
    grid = 40
    blk = N_EDGES // grid
    return pl.pallas_call(
        _tc_body,
        grid=(grid,),
        in_specs=[
            pl.BlockSpec((N_NODES, D_FEAT), lambda i: (0, 0)),
            pl.BlockSpec((D_FEAT, D_OUT), lambda i: (0, 0)),
            pl.BlockSpec((D_FEAT, D_OUT), lambda i: (0, 0)),
            pl.BlockSpec((1, D_OUT), lambda i: (0, 0)),
            pl.BlockSpec((blk, D_EDGE), lambda i: (i, 0)),
            pl.BlockSpec((D_EDGE, D_OUT), lambda i: (0, 0)),
        ],
        out_specs=[
            pl.BlockSpec((N_NODES, D_OUT), lambda i: (0, 0)),
            pl.BlockSpec((N_NODES, D_OUT), lambda i: (0, 0)),
            pl.BlockSpec((blk, D_OUT), lambda i: (i, 0)),
        ],
        out_shape=[
            jax.ShapeDtypeStruct((N_NODES, D_OUT), jnp.float32),
            jax.ShapeDtypeStruct((N_NODES, D_OUT), jnp.float32),
            jax.ShapeDtypeStruct((N_EDGES, D_OUT), jnp.float32),
        ],
    )(x, w1, w3, b2d, edge_attr, w2)


def _sc_body(p1_hbm, p3_hbm, src_hbm, dst_hbm, ea_hbm, out_hbm,
             idx1_v, idx3_v,
             g1_0, g1_1, g3_0, g3_1, acc_0, acc_1, ob_0, ob_1,
             gsem0, gsem1, easem0, easem1, osem0, osem1):
    wid = lax.axis_index("s") * NC + lax.axis_index("c")
    w_base = wid * E_PER_W

    g1 = (g1_0, g1_1)
    g3 = (g3_0, g3_1)
    acc = (acc_0, acc_1)
    ob = (ob_0, ob_1)
    gsem = (gsem0, gsem1)
    easem = (easem0, easem1)
    osem = (osem0, osem1)

    pltpu.sync_copy(src_hbm.at[pl.ds(w_base, E_PER_W)], idx1_v)
    pltpu.sync_copy(dst_hbm.at[pl.ds(w_base, E_PER_W)], idx3_v)

    def in_descs(c, b):
        base = w_base + c * CHUNK
        lb = c * CHUNK
        return (
            pltpu.make_async_copy(
                p1_hbm.at[idx1_v.at[pl.ds(lb, CHUNK)]], g1[b], gsem[b]),
            pltpu.make_async_copy(
                p3_hbm.at[idx3_v.at[pl.ds(lb, CHUNK)]], g3[b], gsem[b]),
            pltpu.make_async_copy(
                ea_hbm.at[pl.ds(base, CHUNK)], acc[b], easem[b]),
        )

    def out_desc(c, b):
        base = w_base + c * CHUNK
        return pltpu.make_async_copy(
            ob[b], out_hbm.at[pl.ds(base, CHUNK)], osem[b])

    def issue(c, b):
        for d in in_descs(c, b):
            d.start()

    def wait_in(c, b):
        for d in in_descs(c, b):
            d.wait()

    def compute(b):
        def row_body(r, _):
            for j in range(D_OUT // 16):
                sl = pl.ds(j * 16, 16)
                ob[b][r, sl] = acc[b][r, sl] + g1[b][r, sl] + g3[b][r, sl]
            return 0

        lax.fori_loop(0, CHUNK, row_body, 0)

    def step(c, b, do_wait_out, do_issue_next):
        wait_in(c, b)
        if do_wait_out:
            out_desc(c - 2, b).wait()
        compute(b)
        if do_issue_next:
            issue(c + 2, b)
        out_desc(c, b).start()

    issue(0, 0)
    issue(1, 1)
    step(0, 0, False, True)
    step(1, 1, False, True)

    def pair_body(i, _):
        step(2 * i, 0, True, True)
        step(2 * i + 1, 1, True, True)
        return 0

    lax.fori_loop(1, (N_CHUNKS - 3) // 2, pair_body, 0)

    step(N_CHUNKS - 3, 0, True, True)
    step(N_CHUNKS - 2, 1, True, False)
    step(N_CHUNKS - 1, 0, True, False)
    out_desc(N_CHUNKS - 2, 1).wait()
    out_desc(N_CHUNKS - 1, 0).wait()


def _sc_combine(p1, p3, src, dst, ea):
    mesh = plsc.VectorSubcoreMesh(core_axis_name="c", subcore_axis_name="s")
    blk = lambda: pltpu.VMEM((CHUNK, D_OUT), jnp.float32)
    f = functools.partial(
        pl.kernel,
        mesh=mesh,
        out_type=jax.ShapeDtypeStruct((N_EDGES, D_OUT), jnp.float32),
        scratch_types=[
            pltpu.VMEM((E_PER_W,), jnp.int32),
            pltpu.VMEM((E_PER_W,), jnp.int32),
            blk(), blk(), blk(), blk(), blk(), blk(), blk(), blk(),
            pltpu.SemaphoreType.DMA,
            pltpu.SemaphoreType.DMA,
            pltpu.SemaphoreType.DMA,
            pltpu.SemaphoreType.DMA,
            pltpu.SemaphoreType.DMA,
            pltpu.SemaphoreType.DMA,
        ],
    )(_sc_body)
    return f(p1, p3, src, dst, ea)


@jax.jit
def kernel(x, edge_attr, edge_index, W, b):
    w1 = W[:D_FEAT]
    w2 = W[D_FEAT:D_FEAT + D_EDGE]
    w3 = W[D_FEAT + D_EDGE:]
    b2d = b.reshape(1, D_OUT)
    p1, p3, ea = _tc_stage(x, w1, w3, b2d, edge_attr, w2)
    return _sc_combine(p1, p3, edge_index[0], edge_index[1], ea)


# speedup vs baseline: 1.7518x; 1.1273x over previous
import functools

import jax
import jax.numpy as jnp
import numpy as np
from jax import lax
from jax.experimental import pallas as pl
from jax.experimental.pallas import tpu as pltpu
from jax.experimental.pallas import tpu_sc as plsc

N_NODES = 10000
N_EDGES = 320000
E_HALF = N_EDGES // 2
D_FEAT = 128
D_EDGE = 16
D_OUT = 128

NC = 2
NS = 16
NW = NC * NS
R_PER_W = E_HALF // NW
CHUNK = 40
N_CHUNKS = R_PER_W // CHUNK

_HI_MASK = np.int32(-65536)


def _round_bits(v):
    return lax.bitcast_convert_type(
        v.astype(jnp.bfloat16).astype(jnp.float32), jnp.int32)


def _pack(vlo, vhi):
    return lax.shift_right_logical(_round_bits(vlo), 16) | (
        _round_bits(vhi) & _HI_MASK)


def _tc_body(x_ref, w1_ref, w3_ref, b_ref, ea_lo_ref, ea_hi_ref, w2_ref,
             p1_ref, p3_ref, eap_ref):
    @pl.when(pl.program_id(0) == 0)
    def _():
        xb = x_ref[...]
        p1_ref[...] = (
            jnp.dot(xb, w1_ref[...], preferred_element_type=jnp.float32)
            + b_ref[...]
        )
        p3_ref[...] = jnp.dot(
            xb, w3_ref[...], preferred_element_type=jnp.float32)

    w2 = w2_ref[...]
    eap_ref[...] = _pack(
        jnp.dot(ea_lo_ref[...], w2, preferred_element_type=jnp.float32),
        jnp.dot(ea_hi_ref[...], w2, preferred_element_type=jnp.float32),
    )


def _tc_stage(x, w1, w3, b2d, edge_attr, w2):
    grid = 20
    blk = E_HALF // grid
    return pl.pallas_call(
        _tc_body,
        grid=(grid,),
        in_specs=[
            pl.BlockSpec((N_NODES, D_FEAT), lambda i: (0, 0)),
            pl.BlockSpec((D_FEAT, D_OUT), lambda i: (0, 0)),
            pl.BlockSpec((D_FEAT, D_OUT), lambda i: (0, 0)),
            pl.BlockSpec((1, D_OUT), lambda i: (0, 0)),
            pl.BlockSpec((blk, D_EDGE), lambda i: (i, 0)),
            pl.BlockSpec((blk, D_EDGE), lambda i: (i + 20, 0)),
            pl.BlockSpec((D_EDGE, D_OUT), lambda i: (0, 0)),
        ],
        out_specs=[
            pl.BlockSpec((N_NODES, D_OUT), lambda i: (0, 0)),
            pl.BlockSpec((N_NODES, D_OUT), lambda i: (0, 0)),
            pl.BlockSpec((blk, D_OUT), lambda i: (i, 0)),
        ],
        out_shape=[
            jax.ShapeDtypeStruct((N_NODES, D_OUT), jnp.float32),
            jax.ShapeDtypeStruct((N_NODES, D_OUT), jnp.float32),
            jax.ShapeDtypeStruct((E_HALF, D_OUT), jnp.int32),
        ],
    )(x, w1, w3, b2d, edge_attr, edge_attr, w2)


def _sc_body(p1_hbm, p3_hbm, src_hbm, dst_hbm, eap_hbm, out_hbm,
             i1lo_v, i3lo_v, i1hi_v, i3hi_v,
             g1lo_0, g1lo_1, g3lo_0, g3lo_1, g1hi_0, g1hi_1, g3hi_0, g3hi_1,
             acc_0, acc_1, oblo_0, oblo_1, obhi_0, obhi_1,
             gsem0, gsem1, easem0, easem1, osem0, osem1):
    wid = lax.axis_index("s") * NC + lax.axis_index("c")
    w_base = wid * R_PER_W

    g1lo = (g1lo_0, g1lo_1)
    g3lo = (g3lo_0, g3lo_1)
    g1hi = (g1hi_0, g1hi_1)
    g3hi = (g3hi_0, g3hi_1)
    acc = (acc_0, acc_1)
    oblo = (oblo_0, oblo_1)
    obhi = (obhi_0, obhi_1)
    gsem = (gsem0, gsem1)
    easem = (easem0, easem1)
    osem = (osem0, osem1)

    pltpu.sync_copy(src_hbm.at[pl.ds(w_base, R_PER_W)], i1lo_v)
    pltpu.sync_copy(dst_hbm.at[pl.ds(w_base, R_PER_W)], i3lo_v)
    pltpu.sync_copy(src_hbm.at[pl.ds(E_HALF + w_base, R_PER_W)], i1hi_v)
    pltpu.sync_copy(dst_hbm.at[pl.ds(E_HALF + w_base, R_PER_W)], i3hi_v)

    def in_descs(c, b):
        base = w_base + c * CHUNK
        lb = c * CHUNK
        return (
            pltpu.make_async_copy(
                p1_hbm.at[i1lo_v.at[pl.ds(lb, CHUNK)]], g1lo[b], gsem[b]),
            pltpu.make_async_copy(
                p3_hbm.at[i3lo_v.at[pl.ds(lb, CHUNK)]], g3lo[b], gsem[b]),
            pltpu.make_async_copy(
                p1_hbm.at[i1hi_v.at[pl.ds(lb, CHUNK)]], g1hi[b], gsem[b]),
            pltpu.make_async_copy(
                p3_hbm.at[i3hi_v.at[pl.ds(lb, CHUNK)]], g3hi[b], gsem[b]),
            pltpu.make_async_copy(
                eap_hbm.at[pl.ds(base, CHUNK)], acc[b], easem[b]),
        )

    def out_descs(c, b):
        base = w_base + c * CHUNK
        return (
            pltpu.make_async_copy(
                oblo[b], out_hbm.at[pl.ds(base, CHUNK)], osem[b]),
            pltpu.make_async_copy(
                obhi[b], out_hbm.at[pl.ds(E_HALF + base, CHUNK)], osem[b]),
        )

    def issue(c, b):
        for d in in_descs(c, b):
            d.start()

    def wait_in(c, b):
        for d in in_descs(c, b):
            d.wait()

    def out_start(c, b):
        for d in out_descs(c, b):
            d.start()

    def out_wait(c, b):
        for d in out_descs(c, b):
            d.wait()

    def _lo(w):
        return lax.bitcast_convert_type(w << 16, jnp.float32)

    def _hi(w):
        return lax.bitcast_convert_type(w & _HI_MASK, jnp.float32)

    def compute(b):
        def row_body(r, _):
            for j in range(D_OUT // 16):
                sl = pl.ds(j * 16, 16)
                w = acc[b][r, sl]
                oblo[b][r, sl] = g1lo[b][r, sl] + g3lo[b][r, sl] + _lo(w)
                obhi[b][r, sl] = g1hi[b][r, sl] + g3hi[b][r, sl] + _hi(w)
            return 0

        lax.fori_loop(0, CHUNK, row_body, 0)

    def step(c, b, do_wait_out, do_issue_next):
        wait_in(c, b)
        if do_wait_out:
            out_wait(c - 2, b)
        compute(b)
        if do_issue_next:
            issue(c + 2, b)
        out_start(c, b)

    issue(0, 0)
    issue(1, 1)
    step(0, 0, False, True)
    step(1, 1, False, True)

    def pair_body(i, _):
        step(2 * i, 0, True, True)
        step(2 * i + 1, 1, True, True)
        return 0

    lax.fori_loop(1, (N_CHUNKS - 3) // 2, pair_body, 0)

    step(N_CHUNKS - 3, 0, True, True)
    step(N_CHUNKS - 2, 1, True, False)
    step(N_CHUNKS - 1, 0, True, False)
    out_wait(N_CHUNKS - 2, 1)
    out_wait(N_CHUNKS - 1, 0)


def _sc_combine(p1, p3, src, dst, eap):
    mesh = plsc.VectorSubcoreMesh(core_axis_name="c", subcore_axis_name="s")
    fblk = lambda: pltpu.VMEM((CHUNK, D_OUT), jnp.float32)
    iblk = lambda: pltpu.VMEM((CHUNK, D_OUT), jnp.int32)
    idxb = lambda: pltpu.VMEM((R_PER_W,), jnp.int32)
    f = functools.partial(
        pl.kernel,
        mesh=mesh,
        out_type=jax.ShapeDtypeStruct((N_EDGES, D_OUT), jnp.float32),
        scratch_types=[
            idxb(), idxb(), idxb(), idxb(),
            fblk(), fblk(), fblk(), fblk(),
            fblk(), fblk(), fblk(), fblk(),
            iblk(), iblk(),
            fblk(), fblk(), fblk(), fblk(),
            pltpu.SemaphoreType.DMA,
            pltpu.SemaphoreType.DMA,
            pltpu.SemaphoreType.DMA,
            pltpu.SemaphoreType.DMA,
            pltpu.SemaphoreType.DMA,
            pltpu.SemaphoreType.DMA,
        ],
    )(_sc_body)
    return f(p1, p3, src, dst, eap)


@jax.jit
def kernel(x, edge_attr, edge_index, W, b):
    w1 = W[:D_FEAT]
    w2 = W[D_FEAT:D_FEAT + D_EDGE]
    w3 = W[D_FEAT + D_EDGE:]
    b2d = b.reshape(1, D_OUT)
    p1, p3, eap = _tc_stage(x, w1, w3, b2d, edge_attr, w2)
    return _sc_combine(p1, p3, edge_index[0], edge_index[1], eap)
